# baseline (speedup 1.0000x reference)
import functools

import jax
import jax.numpy as jnp
from jax import lax
from jax.experimental import pallas as pl
from jax.experimental.pallas import tpu as pltpu
from jax.experimental.pallas import tpu_sc as plsc

_NC = 2
_NS = 16
_LANES = 16
_DEGW = 16
_CH = 80


@functools.lru_cache(maxsize=None)
def _make_sc_hop(N, D, E, with_deg):
  NW = _NC * _NS
  e_per_w = E // NW
  n_ch = e_per_w // _CH
  rows_per_tile = N // _NS

  mesh = plsc.VectorSubcoreMesh(core_axis_name="c", subcore_axis_name="s")
  out_type = [jax.ShapeDtypeStruct((_NC, N, D), jnp.float32)]
  scratch = [
      pltpu.VMEM((e_per_w,), jnp.int32),
      pltpu.VMEM((e_per_w,), jnp.int32),
      pltpu.VMEM((_CH,), jnp.int32),
      pltpu.VMEM((_CH, D), jnp.float32),
      pltpu.VMEM((_CH,), jnp.int32),
      pltpu.VMEM((_CH, D), jnp.float32),
      pltpu.VMEM_SHARED((N, D), jnp.float32),
      pltpu.SemaphoreType.DMA,
      pltpu.SemaphoreType.DMA,
      pltpu.SemaphoreType.DMA,
      pltpu.SemaphoreType.DMA,
  ]
  if with_deg:
    out_type.append(jax.ShapeDtypeStruct((_NC * N,), jnp.float32))
    scratch += [
        pltpu.VMEM((_CH,), jnp.float32),
        pltpu.VMEM_SHARED((N,), jnp.float32),
        pltpu.VMEM((rows_per_tile,), jnp.float32),
    ]

  def body(h_hbm, src_hbm, dst_hbm, zeros_hbm, zdeg_hbm, ones_hbm, *refs):
    if with_deg:
      (agg_out, deg_out, src_v, dst_v, dstc0, rows0, dstc1, rows1, agg_sh,
       sem0, sem1, ssem0, ssem1, ones_v, deg_sh, degbuf) = refs
    else:
      (agg_out, src_v, dst_v, dstc0, rows0, dstc1, rows1, agg_sh,
       sem0, sem1, ssem0, ssem1) = refs
    bufs = ((dstc0, rows0, sem0, ssem0), (dstc1, rows1, sem1, ssem1))
    cid = lax.axis_index("c")
    sid = lax.axis_index("s")
    wid = sid * _NC + cid
    r0 = sid * rows_per_tile

    pltpu.sync_copy(zeros_hbm.at[pl.ds(r0, rows_per_tile)],
                    agg_sh.at[pl.ds(r0, rows_per_tile)])
    if with_deg:
      pltpu.sync_copy(zdeg_hbm.at[pl.ds(r0, rows_per_tile)], degbuf)
      pltpu.sync_copy(degbuf, deg_sh.at[pl.ds(r0, rows_per_tile)])
      pltpu.sync_copy(ones_hbm, ones_v)

    base = wid * e_per_w
    pltpu.sync_copy(src_hbm.at[pl.ds(base, e_per_w)], src_v)
    pltpu.sync_copy(dst_hbm.at[pl.ds(base, e_per_w)], dst_v)
    plsc.subcore_barrier()

    def start(i, buf):
      dstcb, rowsb, semb, _ = buf
      off = i * _CH
      for j in range(_CH // _LANES):
        dstcb[pl.ds(j * _LANES, _LANES)] = dst_v[pl.ds(off + j * _LANES,
                                                       _LANES)]
      pltpu.async_copy(h_hbm.at[src_v.at[pl.ds(off, _CH)]], rowsb, semb)

    def wait_gather(buf):
      dstcb, rowsb, semb, _ = buf
      pltpu.make_async_copy(h_hbm.at[dstcb], rowsb, semb).wait()

    def issue_scatter(buf):
      dstcb, rowsb, _, ssemb = buf
      pltpu.async_copy(rowsb, agg_sh.at[dstcb], ssemb, add=True)
      if with_deg:
        pltpu.async_copy(ones_v, deg_sh.at[dstcb], ssemb, add=True)

    def wait_scatter(buf):
      dstcb, rowsb, _, ssemb = buf
      pltpu.make_async_copy(rowsb, agg_sh.at[dstcb], ssemb).wait()
      if with_deg:
        pltpu.make_async_copy(ones_v, deg_sh.at[dstcb], ssemb).wait()

    assert n_ch % 2 == 1 and n_ch >= 3
    npairs = (n_ch - 1) // 2
    start(0, bufs[0])
    start(1, bufs[1])

    def pair(i, c):
      wait_gather(bufs[0])
      issue_scatter(bufs[0])
      wait_gather(bufs[1])
      issue_scatter(bufs[1])
      wait_scatter(bufs[0])
      start(2 * i + 2, bufs[0])
      wait_scatter(bufs[1])

      @pl.when(i < npairs - 1)
      def _():
        start(2 * i + 3, bufs[1])

      return c

    lax.fori_loop(0, npairs, pair, 0)
    wait_gather(bufs[0])
    issue_scatter(bufs[0])
    wait_scatter(bufs[0])
    plsc.subcore_barrier()

    pltpu.sync_copy(agg_sh.at[pl.ds(r0, rows_per_tile)],
                    agg_out.at[cid, pl.ds(r0, rows_per_tile)])
    if with_deg:
      pltpu.sync_copy(deg_sh.at[pl.ds(r0, rows_per_tile)], degbuf)
      pltpu.sync_copy(degbuf, deg_out.at[pl.ds(cid * N + r0, rows_per_tile)])

  return pl.kernel(body, out_type=tuple(out_type) if with_deg else out_type[0],
                   mesh=mesh, scratch_types=scratch)


def _elu(x):
  return jnp.where(x > 0, x, jnp.exp(x) - 1.0)


def _dense_body(aggp, degp, w, b, prim, out):
  a = aggp[0] + aggp[1]
  deg1 = jnp.maximum(degp[0] + degp[1], 1.0)
  x = jnp.dot(a / deg1, w[...], preferred_element_type=jnp.float32) + b[...]
  out[...] = _elu(x) + prim[...]


def _dense_gru_body(aggp, degp, w, b, prim, hid, wit, wht, bi, bh, out):
  a = aggp[0] + aggp[1]
  deg1 = jnp.maximum(degp[0] + degp[1], 1.0)
  x = jnp.dot(a / deg1, w[...], preferred_element_type=jnp.float32) + b[...]
  s = _elu(x) + prim[...]
  h = hid[...]
  gi = jnp.dot(s, wit[...], preferred_element_type=jnp.float32) + bi[...]
  gh = jnp.dot(h, wht[...], preferred_element_type=jnp.float32) + bh[...]
  D = s.shape[1]
  r = jax.nn.sigmoid(gi[:, :D] + gh[:, :D])
  z = jax.nn.sigmoid(gi[:, D:2 * D] + gh[:, D:2 * D])
  n = jnp.tanh(gi[:, 2 * D:] + r * gh[:, 2 * D:])
  out[...] = (1.0 - z) * n + z * h


@functools.lru_cache(maxsize=None)
def _make_dense(N, D, B):
  grid = (N // B,)
  return pl.pallas_call(
      _dense_body,
      grid=grid,
      in_specs=[
          pl.BlockSpec((2, B, D), lambda i: (0, i, 0)),
          pl.BlockSpec((2, B, 1), lambda i: (0, i, 0)),
          pl.BlockSpec((D, D), lambda i: (0, 0)),
          pl.BlockSpec((1, D), lambda i: (0, 0)),
          pl.BlockSpec((B, D), lambda i: (i, 0)),
      ],
      out_specs=pl.BlockSpec((B, D), lambda i: (i, 0)),
      out_shape=jax.ShapeDtypeStruct((N, D), jnp.float32),
  )


@functools.lru_cache(maxsize=None)
def _make_dense_gru(N, D, B):
  grid = (N // B,)
  return pl.pallas_call(
      _dense_gru_body,
      grid=grid,
      in_specs=[
          pl.BlockSpec((2, B, D), lambda i: (0, i, 0)),
          pl.BlockSpec((2, B, 1), lambda i: (0, i, 0)),
          pl.BlockSpec((D, D), lambda i: (0, 0)),
          pl.BlockSpec((1, D), lambda i: (0, 0)),
          pl.BlockSpec((B, D), lambda i: (i, 0)),
          pl.BlockSpec((B, D), lambda i: (i, 0)),
          pl.BlockSpec((D, 3 * D), lambda i: (0, 0)),
          pl.BlockSpec((D, 3 * D), lambda i: (0, 0)),
          pl.BlockSpec((1, 3 * D), lambda i: (0, 0)),
          pl.BlockSpec((1, 3 * D), lambda i: (0, 0)),
      ],
      out_specs=pl.BlockSpec((B, D), lambda i: (i, 0)),
      out_shape=jax.ShapeDtypeStruct((N, D), jnp.float32),
  )


def kernel(k, edge_index, primary_emb, W_hops, b_hops, gru_Wi, gru_Wh,
           gru_bi, gru_bh):
  del k
  T = edge_index.shape[0]
  E = edge_index.shape[2]
  N, D = primary_emb.shape
  hops = W_hops.shape[0] - 1
  B = 2000
  npad = 8 * _NS
  N_PAD = ((N + npad - 1) // npad) * npad

  sc_hop_deg = _make_sc_hop(N_PAD, D, E, True)
  sc_hop = _make_sc_hop(N_PAD, D, E, False)
  dense = _make_dense(N, D, B)
  dense_gru = _make_dense_gru(N, D, B)

  zeros = jnp.zeros((N_PAD, D), jnp.float32)
  zdeg = jnp.zeros((N_PAD,), jnp.float32)
  ones_rows = jnp.ones((_CH,), jnp.float32)
  WiT = gru_Wi.T
  WhT = gru_Wh.T
  bi = gru_bi[None, :]
  bh = gru_bh[None, :]
  ei = edge_index.astype(jnp.int32)

  hidden = None
  for t in range(T):
    src = ei[t, 0]
    dst = ei[t, 1]
    h = primary_emb
    degp = None
    for hop in range(hops):
      w = W_hops[hop]
      b = b_hops[hop][None, :]
      if hop == 0:
        aggp, degp = sc_hop_deg(h, src, dst, zeros, zdeg, ones_rows)
        degp = degp.reshape(_NC, N_PAD, 1)
      else:
        aggp = sc_hop(h, src, dst, zeros, zdeg, ones_rows)
      last = hop == hops - 1
      if last and hidden is not None:
        hidden = dense_gru(aggp, degp, w, b, primary_emb, hidden,
                           WiT, WhT, bi, bh)
      else:
        h = dense(aggp, degp, w, b, primary_emb)
        if last:
          hidden = h
  return hidden


# speedup vs baseline: 1.2535x; 1.2535x over previous
import functools

import jax
import jax.numpy as jnp
from jax import lax
from jax.experimental import pallas as pl
from jax.experimental.pallas import tpu as pltpu
from jax.experimental.pallas import tpu_sc as plsc

_NC = 2
_NS = 16
_LANES = 16
_DEGW = 16
_CH = 80


@functools.lru_cache(maxsize=None)
def _make_sc_hop(N, D, E, with_deg):
  NW = _NC * _NS
  e_per_w = E // NW
  n_ch = e_per_w // _CH
  rows_per_tile = N // _NS

  mesh = plsc.VectorSubcoreMesh(core_axis_name="c", subcore_axis_name="s")
  out_type = [jax.ShapeDtypeStruct((_NC, N, D), jnp.float32)]
  scratch = [
      pltpu.VMEM((e_per_w,), jnp.int32),
      pltpu.VMEM((e_per_w,), jnp.int32),
      pltpu.VMEM((_CH,), jnp.int32),
      pltpu.VMEM((_CH, D), jnp.float32),
      pltpu.VMEM((_CH,), jnp.int32),
      pltpu.VMEM((_CH, D), jnp.float32),
      pltpu.VMEM_SHARED((N, D), jnp.float32),
      pltpu.SemaphoreType.DMA,
      pltpu.SemaphoreType.DMA,
  ]
  if with_deg:
    out_type.append(jax.ShapeDtypeStruct((_NC * N,), jnp.float32))
    scratch += [
        pltpu.VMEM((_CH,), jnp.float32),
        pltpu.VMEM_SHARED((N,), jnp.float32),
        pltpu.VMEM((rows_per_tile,), jnp.float32),
    ]

  def body(h_hbm, src_hbm, dst_hbm, zeros_hbm, zdeg_hbm, ones_hbm, *refs):
    if with_deg:
      (agg_out, deg_out, src_v, dst_v, dstc0, rows0, dstc1, rows1, agg_sh,
       sem0, sem1, ones_v, deg_sh, degbuf) = refs
    else:
      (agg_out, src_v, dst_v, dstc0, rows0, dstc1, rows1, agg_sh,
       sem0, sem1) = refs
    bufs = ((dstc0, rows0, sem0), (dstc1, rows1, sem1))
    cid = lax.axis_index("c")
    sid = lax.axis_index("s")
    wid = sid * _NC + cid
    r0 = sid * rows_per_tile

    pltpu.sync_copy(zeros_hbm.at[pl.ds(r0, rows_per_tile)],
                    agg_sh.at[pl.ds(r0, rows_per_tile)])
    if with_deg:
      pltpu.sync_copy(zdeg_hbm.at[pl.ds(r0, rows_per_tile)], degbuf)
      pltpu.sync_copy(degbuf, deg_sh.at[pl.ds(r0, rows_per_tile)])
      pltpu.sync_copy(ones_hbm, ones_v)

    base = wid * e_per_w
    pltpu.sync_copy(src_hbm.at[pl.ds(base, e_per_w)], src_v)
    pltpu.sync_copy(dst_hbm.at[pl.ds(base, e_per_w)], dst_v)
    plsc.subcore_barrier()

    def start(i, buf):
      dstcb, rowsb, semb = buf
      off = i * _CH
      for j in range(_CH // _LANES):
        dstcb[pl.ds(j * _LANES, _LANES)] = dst_v[pl.ds(off + j * _LANES,
                                                       _LANES)]
      pltpu.async_copy(h_hbm.at[src_v.at[pl.ds(off, _CH)]], rowsb, semb)

    def finish(buf):
      dstcb, rowsb, semb = buf
      pltpu.make_async_copy(h_hbm.at[dstcb], rowsb, semb).wait()
      pltpu.sync_copy(rowsb, agg_sh.at[dstcb], add=True)
      if with_deg:
        pltpu.sync_copy(ones_v, deg_sh.at[dstcb], add=True)

    assert n_ch % 2 == 1 and n_ch >= 3
    npairs = (n_ch - 1) // 2
    start(0, bufs[0])
    start(1, bufs[1])

    def pair(i, c):
      finish(bufs[0])
      start(2 * i + 2, bufs[0])
      finish(bufs[1])

      @pl.when(i < npairs - 1)
      def _():
        start(2 * i + 3, bufs[1])

      return c

    lax.fori_loop(0, npairs, pair, 0)
    finish(bufs[0])
    plsc.subcore_barrier()

    pltpu.sync_copy(agg_sh.at[pl.ds(r0, rows_per_tile)],
                    agg_out.at[cid, pl.ds(r0, rows_per_tile)])
    if with_deg:
      pltpu.sync_copy(deg_sh.at[pl.ds(r0, rows_per_tile)], degbuf)
      pltpu.sync_copy(degbuf, deg_out.at[pl.ds(cid * N + r0, rows_per_tile)])

  return pl.kernel(body, out_type=tuple(out_type) if with_deg else out_type[0],
                   mesh=mesh, scratch_types=scratch)


def _elu(x):
  return jnp.where(x > 0, x, jnp.exp(x) - 1.0)


def _dense_body(aggp, degp, w, b, prim, out):
  a = aggp[0] + aggp[1]
  deg1 = jnp.maximum(degp[0] + degp[1], 1.0)
  x = jnp.dot(a / deg1, w[...], preferred_element_type=jnp.float32) + b[...]
  out[...] = _elu(x) + prim[...]


def _dense_gru_body(aggp, degp, w, b, prim, hid, wit, wht, bi, bh, out):
  a = aggp[0] + aggp[1]
  deg1 = jnp.maximum(degp[0] + degp[1], 1.0)
  x = jnp.dot(a / deg1, w[...], preferred_element_type=jnp.float32) + b[...]
  s = _elu(x) + prim[...]
  h = hid[...]
  gi = jnp.dot(s, wit[...], preferred_element_type=jnp.float32) + bi[...]
  gh = jnp.dot(h, wht[...], preferred_element_type=jnp.float32) + bh[...]
  D = s.shape[1]
  r = jax.nn.sigmoid(gi[:, :D] + gh[:, :D])
  z = jax.nn.sigmoid(gi[:, D:2 * D] + gh[:, D:2 * D])
  n = jnp.tanh(gi[:, 2 * D:] + r * gh[:, 2 * D:])
  out[...] = (1.0 - z) * n + z * h


@functools.lru_cache(maxsize=None)
def _make_dense(N, D, B):
  grid = (N // B,)
  return pl.pallas_call(
      _dense_body,
      grid=grid,
      in_specs=[
          pl.BlockSpec((2, B, D), lambda i: (0, i, 0)),
          pl.BlockSpec((2, B, 1), lambda i: (0, i, 0)),
          pl.BlockSpec((D, D), lambda i: (0, 0)),
          pl.BlockSpec((1, D), lambda i: (0, 0)),
          pl.BlockSpec((B, D), lambda i: (i, 0)),
      ],
      out_specs=pl.BlockSpec((B, D), lambda i: (i, 0)),
      out_shape=jax.ShapeDtypeStruct((N, D), jnp.float32),
  )


@functools.lru_cache(maxsize=None)
def _make_dense_gru(N, D, B):
  grid = (N // B,)
  return pl.pallas_call(
      _dense_gru_body,
      grid=grid,
      in_specs=[
          pl.BlockSpec((2, B, D), lambda i: (0, i, 0)),
          pl.BlockSpec((2, B, 1), lambda i: (0, i, 0)),
          pl.BlockSpec((D, D), lambda i: (0, 0)),
          pl.BlockSpec((1, D), lambda i: (0, 0)),
          pl.BlockSpec((B, D), lambda i: (i, 0)),
          pl.BlockSpec((B, D), lambda i: (i, 0)),
          pl.BlockSpec((D, 3 * D), lambda i: (0, 0)),
          pl.BlockSpec((D, 3 * D), lambda i: (0, 0)),
          pl.BlockSpec((1, 3 * D), lambda i: (0, 0)),
          pl.BlockSpec((1, 3 * D), lambda i: (0, 0)),
      ],
      out_specs=pl.BlockSpec((B, D), lambda i: (i, 0)),
      out_shape=jax.ShapeDtypeStruct((N, D), jnp.float32),
  )


def kernel(k, edge_index, primary_emb, W_hops, b_hops, gru_Wi, gru_Wh,
           gru_bi, gru_bh):
  del k
  T = edge_index.shape[0]
  E = edge_index.shape[2]
  N, D = primary_emb.shape
  hops = W_hops.shape[0] - 1
  B = 2000
  npad = 8 * _NS
  N_PAD = ((N + npad - 1) // npad) * npad

  sc_hop_deg = _make_sc_hop(N_PAD, D, E, True)
  sc_hop = _make_sc_hop(N_PAD, D, E, False)
  dense = _make_dense(N, D, B)
  dense_gru = _make_dense_gru(N, D, B)

  zeros = jnp.zeros((N_PAD, D), jnp.float32)
  zdeg = jnp.zeros((N_PAD,), jnp.float32)
  ones_rows = jnp.ones((_CH,), jnp.float32)
  WiT = gru_Wi.T
  WhT = gru_Wh.T
  bi = gru_bi[None, :]
  bh = gru_bh[None, :]
  ei = edge_index.astype(jnp.int32)

  hidden = None
  for t in range(T):
    src = ei[t, 0]
    dst = ei[t, 1]
    h = primary_emb
    degp = None
    for hop in range(hops):
      w = W_hops[hop]
      b = b_hops[hop][None, :]
      if hop == 0:
        aggp, degp = sc_hop_deg(h, src, dst, zeros, zdeg, ones_rows)
        degp = degp.reshape(_NC, N_PAD, 1)
      else:
        aggp = sc_hop(h, src, dst, zeros, zdeg, ones_rows)
      last = hop == hops - 1
      if last and hidden is not None:
        hidden = dense_gru(aggp, degp, w, b, primary_emb, hidden,
                           WiT, WhT, bi, bh)
      else:
        h = dense(aggp, degp, w, b, primary_emb)
        if last:
          hidden = h
  return hidden


# speedup vs baseline: 1.2548x; 1.0010x over previous
import functools

import jax
import jax.numpy as jnp
from jax import lax
from jax.experimental import pallas as pl
from jax.experimental.pallas import tpu as pltpu
from jax.experimental.pallas import tpu_sc as plsc

_NC = 2
_NS = 16
_LANES = 16
_CH = 80


@functools.lru_cache(maxsize=None)
def _make_sc_hop(N, D, E, with_deg):
  NW = _NC * _NS
  e_per_w = E // NW
  n_ch = e_per_w // _CH
  rows_per_tile = N // _NS

  mesh = plsc.VectorSubcoreMesh(core_axis_name="c", subcore_axis_name="s")
  out_type = [jax.ShapeDtypeStruct((_NC, N, D), jnp.float32)]
  scratch = [
      pltpu.VMEM((e_per_w,), jnp.int32),
      pltpu.VMEM((e_per_w,), jnp.int32),
      pltpu.VMEM((_CH,), jnp.int32),
      pltpu.VMEM((_CH, D), jnp.float32),
      pltpu.VMEM((_CH,), jnp.int32),
      pltpu.VMEM((_CH, D), jnp.float32),
      pltpu.VMEM_SHARED((N, D), jnp.float32),
      pltpu.SemaphoreType.DMA,
      pltpu.SemaphoreType.DMA,
  ]
  if with_deg:
    out_type.append(jax.ShapeDtypeStruct((_NC * N,), jnp.float32))
    scratch += [
        pltpu.VMEM((_CH,), jnp.float32),
        pltpu.VMEM_SHARED((N,), jnp.float32),
        pltpu.VMEM((rows_per_tile,), jnp.float32),
    ]

  def body(h_hbm, src_hbm, dst_hbm, zeros_hbm, zdeg_hbm, ones_hbm, *refs):
    if with_deg:
      (agg_out, deg_out, src_v, dst_v, dstc0, rows0, dstc1, rows1, agg_sh,
       sem0, sem1, ones_v, deg_sh, degbuf) = refs
    else:
      (agg_out, src_v, dst_v, dstc0, rows0, dstc1, rows1, agg_sh,
       sem0, sem1) = refs
    bufs = ((dstc0, rows0, sem0), (dstc1, rows1, sem1))
    cid = lax.axis_index("c")
    sid = lax.axis_index("s")
    wid = sid * _NC + cid
    r0 = sid * rows_per_tile

    pltpu.sync_copy(zeros_hbm.at[pl.ds(r0, rows_per_tile)],
                    agg_sh.at[pl.ds(r0, rows_per_tile)])
    if with_deg:
      pltpu.sync_copy(zdeg_hbm.at[pl.ds(r0, rows_per_tile)], degbuf)
      pltpu.sync_copy(degbuf, deg_sh.at[pl.ds(r0, rows_per_tile)])
      pltpu.sync_copy(ones_hbm, ones_v)

    base = wid * e_per_w
    pltpu.sync_copy(src_hbm.at[pl.ds(base, e_per_w)], src_v)
    pltpu.sync_copy(dst_hbm.at[pl.ds(base, e_per_w)], dst_v)
    plsc.subcore_barrier()

    def start(i, buf):
      dstcb, rowsb, semb = buf
      off = i * _CH
      for j in range(_CH // _LANES):
        dstcb[pl.ds(j * _LANES, _LANES)] = dst_v[pl.ds(off + j * _LANES,
                                                       _LANES)]
      pltpu.async_copy(h_hbm.at[src_v.at[pl.ds(off, _CH)]], rowsb, semb)

    def finish(buf):
      dstcb, rowsb, semb = buf
      pltpu.make_async_copy(h_hbm.at[dstcb], rowsb, semb).wait()
      pltpu.sync_copy(rowsb, agg_sh.at[dstcb], add=True)
      if with_deg:
        pltpu.sync_copy(ones_v, deg_sh.at[dstcb], add=True)

    assert n_ch % 2 == 1 and n_ch >= 3
    npairs = (n_ch - 1) // 2
    start(0, bufs[0])
    start(1, bufs[1])

    def pair(i, c):
      finish(bufs[0])
      start(2 * i + 2, bufs[0])
      finish(bufs[1])

      @pl.when(i < npairs - 1)
      def _():
        start(2 * i + 3, bufs[1])

      return c

    lax.fori_loop(0, npairs, pair, 0)
    finish(bufs[0])
    plsc.subcore_barrier()

    pltpu.sync_copy(agg_sh.at[pl.ds(r0, rows_per_tile)],
                    agg_out.at[cid, pl.ds(r0, rows_per_tile)])
    if with_deg:
      pltpu.sync_copy(deg_sh.at[pl.ds(r0, rows_per_tile)], degbuf)
      pltpu.sync_copy(degbuf, deg_out.at[pl.ds(cid * N + r0, rows_per_tile)])

  return pl.kernel(body, out_type=tuple(out_type) if with_deg else out_type[0],
                   mesh=mesh, scratch_types=scratch)


def _elu(x):
  return jnp.where(x > 0, x, jnp.exp(x) - 1.0)


def _dense_body(aggp, degp, w, b, prim, out):
  a = aggp[0] + aggp[1]
  deg1 = jnp.maximum(degp[0] + degp[1], 1.0)
  x = jnp.dot(a / deg1, w[...], preferred_element_type=jnp.float32) + b[...]
  out[...] = _elu(x) + prim[...]


def _dense_gru_body(aggp, degp, w, b, prim, hid, wit, wht, bi, bh, out):
  a = aggp[0] + aggp[1]
  deg1 = jnp.maximum(degp[0] + degp[1], 1.0)
  x = jnp.dot(a / deg1, w[...], preferred_element_type=jnp.float32) + b[...]
  s = _elu(x) + prim[...]
  h = hid[...]
  gi = jnp.dot(s, wit[...], preferred_element_type=jnp.float32) + bi[...]
  gh = jnp.dot(h, wht[...], preferred_element_type=jnp.float32) + bh[...]
  D = s.shape[1]
  r = jax.nn.sigmoid(gi[:, :D] + gh[:, :D])
  z = jax.nn.sigmoid(gi[:, D:2 * D] + gh[:, D:2 * D])
  n = jnp.tanh(gi[:, 2 * D:] + r * gh[:, 2 * D:])
  out[...] = (1.0 - z) * n + z * h


@functools.lru_cache(maxsize=None)
def _make_dense(N, D, B):
  grid = (N // B,)
  return pl.pallas_call(
      _dense_body,
      grid=grid,
      in_specs=[
          pl.BlockSpec((2, B, D), lambda i: (0, i, 0)),
          pl.BlockSpec((2, B, 1), lambda i: (0, i, 0)),
          pl.BlockSpec((D, D), lambda i: (0, 0)),
          pl.BlockSpec((1, D), lambda i: (0, 0)),
          pl.BlockSpec((B, D), lambda i: (i, 0)),
      ],
      out_specs=pl.BlockSpec((B, D), lambda i: (i, 0)),
      out_shape=jax.ShapeDtypeStruct((N, D), jnp.float32),
  )


@functools.lru_cache(maxsize=None)
def _make_dense_gru(N, D, B):
  grid = (N // B,)
  return pl.pallas_call(
      _dense_gru_body,
      grid=grid,
      in_specs=[
          pl.BlockSpec((2, B, D), lambda i: (0, i, 0)),
          pl.BlockSpec((2, B, 1), lambda i: (0, i, 0)),
          pl.BlockSpec((D, D), lambda i: (0, 0)),
          pl.BlockSpec((1, D), lambda i: (0, 0)),
          pl.BlockSpec((B, D), lambda i: (i, 0)),
          pl.BlockSpec((B, D), lambda i: (i, 0)),
          pl.BlockSpec((D, 3 * D), lambda i: (0, 0)),
          pl.BlockSpec((D, 3 * D), lambda i: (0, 0)),
          pl.BlockSpec((1, 3 * D), lambda i: (0, 0)),
          pl.BlockSpec((1, 3 * D), lambda i: (0, 0)),
      ],
      out_specs=pl.BlockSpec((B, D), lambda i: (i, 0)),
      out_shape=jax.ShapeDtypeStruct((N, D), jnp.float32),
  )


def kernel(k, edge_index, primary_emb, W_hops, b_hops, gru_Wi, gru_Wh,
           gru_bi, gru_bh):
  del k
  T = edge_index.shape[0]
  E = edge_index.shape[2]
  N, D = primary_emb.shape
  hops = W_hops.shape[0] - 1
  B = 2000
  npad = 8 * _NS
  N_PAD = ((N + npad - 1) // npad) * npad

  sc_hop_deg = _make_sc_hop(N_PAD, D, E, True)
  sc_hop = _make_sc_hop(N_PAD, D, E, False)
  dense = _make_dense(N, D, B)
  dense_gru = _make_dense_gru(N, D, B)

  zeros = jnp.zeros((N_PAD, D), jnp.float32)
  zdeg = jnp.zeros((N_PAD,), jnp.float32)
  ones_rows = jnp.ones((_CH,), jnp.float32)
  WiT = gru_Wi.T
  WhT = gru_Wh.T
  bi = gru_bi[None, :]
  bh = gru_bh[None, :]
  ei = edge_index.astype(jnp.int32)

  hidden = None
  for t in range(T):
    src = ei[t, 0]
    dst = ei[t, 1]
    h = primary_emb
    degp = None
    for hop in range(hops):
      w = W_hops[hop]
      b = b_hops[hop][None, :]
      if hop == 0:
        aggp, degp = sc_hop_deg(h, src, dst, zeros, zdeg, ones_rows)
        degp = degp.reshape(_NC, N_PAD, 1)
      else:
        aggp = sc_hop(h, src, dst, zeros, zdeg, ones_rows)
      last = hop == hops - 1
      if last and hidden is not None:
        hidden = dense_gru(aggp, degp, w, b, primary_emb, hidden,
                           WiT, WhT, bi, bh)
      else:
        h = dense(aggp, degp, w, b, primary_emb)
        if last:
          hidden = h
  return hidden
